# baseline (speedup 1.0000x reference)
import functools

import jax
import jax.numpy as jnp
import numpy as np
from jax import lax
from jax.experimental import pallas as pl
from jax.experimental.pallas import tpu as pltpu
from jax.experimental.pallas import tpu_sc as plsc

_NC = 2
_NS = 16
_NW = _NC * _NS
_B = 64
_WB = 24


def _mesh():
  return plsc.VectorSubcoreMesh(core_axis_name="c", subcore_axis_name="s")


def _make_deg(n_pad, nb):
  rpt = n_pad // _NS

  @functools.partial(
      pl.kernel,
      out_type=jax.ShapeDtypeStruct((_NC, n_pad), jnp.float32),
      mesh=_mesh(),
      scratch_types=[
          pltpu.VMEM((nb, _B), jnp.int32),
          pltpu.VMEM((nb, _B), jnp.float32),
          pltpu.VMEM((_B,), jnp.float32),
          pltpu.VMEM_SHARED((n_pad,), jnp.float32),
      ],
  )
  def deg_k(col_hbm, w_hbm, out_hbm, col_v, w_v, zbuf, acc):
    cid = lax.axis_index("c")
    sid = lax.axis_index("s")
    wid = cid * _NS + sid

    def zlane(i, _):
      zbuf[pl.ds(i * 16, 16)] = jnp.zeros((16,), jnp.float32)
      return 0

    lax.fori_loop(0, _B // 16, zlane, 0)

    base = pl.multiple_of(sid * rpt, 8)
    nfull, rem = divmod(rpt, _B)
    for zi in range(nfull):
      pltpu.sync_copy(zbuf, acc.at[pl.ds(base + zi * _B, _B)])
    if rem:
      pltpu.sync_copy(zbuf.at[pl.ds(0, rem)],
                      acc.at[pl.ds(base + nfull * _B, rem)])
    pltpu.sync_copy(col_hbm.at[wid], col_v)
    pltpu.sync_copy(w_hbm.at[wid], w_v)
    plsc.subcore_barrier()

    def body(j, _):
      pltpu.sync_copy(w_v.at[j], acc.at[col_v.at[j]], add=True)
      return 0

    lax.fori_loop(0, nb, body, 0)
    plsc.subcore_barrier()
    pltpu.sync_copy(acc.at[pl.ds(base, rpt)], out_hbm.at[cid, pl.ds(base, rpt)])

  return deg_k


def _dis_tc(deg_parts2d):
  _, r, c = deg_parts2d.shape

  def body(p_ref, o_ref):
    deg = p_ref[0] + p_ref[1]
    o_ref[...] = jnp.where(
        deg > 0.0, lax.rsqrt(jnp.maximum(deg, 1e-12)), 0.0)

  return pl.pallas_call(
      body,
      out_shape=jax.ShapeDtypeStruct((r, c), jnp.float32),
  )(deg_parts2d)


def _make_norm(n_pad, nb):
  @functools.partial(
      pl.kernel,
      out_type=jax.ShapeDtypeStruct((_NW, nb, _B), jnp.float32),
      mesh=_mesh(),
      compiler_params=pltpu.CompilerParams(needs_layout_passes=False),
      scratch_types=[
          pltpu.VMEM((nb, _B), jnp.int32),
          pltpu.VMEM((nb, _B), jnp.int32),
          pltpu.VMEM((nb, _B), jnp.float32),
          pltpu.VMEM((nb, _B), jnp.float32),
          pltpu.VMEM((n_pad,), jnp.float32),
      ],
  )
  def norm_k(row_hbm, col_hbm, w_hbm, dis_hbm, out_hbm,
             row_v, col_v, w_v, norm_v, dis_v):
    cid = lax.axis_index("c")
    sid = lax.axis_index("s")
    wid = cid * _NS + sid
    pltpu.sync_copy(row_hbm.at[wid], row_v)
    pltpu.sync_copy(col_hbm.at[wid], col_v)
    pltpu.sync_copy(w_hbm.at[wid], w_v)
    pltpu.sync_copy(dis_hbm, dis_v)

    def body(j, _):
      for k in range(_B // 16):
        sl = pl.ds(k * 16, 16)
        a = plsc.load_gather(dis_v, [row_v[j, sl]])
        bb = plsc.load_gather(dis_v, [col_v[j, sl]])
        norm_v[j, sl] = a * w_v[j, sl] * bb
      return 0

    lax.fori_loop(0, nb, body, 0)
    pltpu.sync_copy(norm_v, out_hbm.at[wid])

  return norm_k


def _make_hop(n_pad, d, nb):
  rpt = n_pad // _NS
  nwin = nb // _WB

  @functools.partial(
      pl.kernel,
      out_type=jax.ShapeDtypeStruct((_NC, n_pad, d), jnp.float32),
      mesh=_mesh(),
      compiler_params=pltpu.CompilerParams(needs_layout_passes=False),
      scratch_types=[
          pltpu.VMEM((_WB, 2, _B), jnp.int32),
          pltpu.VMEM((_WB, _B), jnp.float32),
          pltpu.VMEM((_B, d), jnp.float32),
          pltpu.VMEM((_B, d), jnp.float32),
          pltpu.VMEM((_B, d), jnp.float32),
          pltpu.VMEM_SHARED((n_pad, d), jnp.float32),
          pltpu.SemaphoreType.DMA,
          pltpu.SemaphoreType.DMA,
          pltpu.SemaphoreType.DMA,
          pltpu.SemaphoreType.DMA,
          pltpu.SemaphoreType.DMA,
          pltpu.SemaphoreType.DMA,
      ],
  )
  def hop_k(h_hbm, idx_hbm, norm_hbm, out_hbm,
            idx_win, norm_win, fbuf0, fbuf1, fbuf2,
            acc, gs0, gs1, gs2, ss0, ss1, ss2):
    cid = lax.axis_index("c")
    sid = lax.axis_index("s")
    wid = cid * _NS + sid
    fbufs = (fbuf0, fbuf1, fbuf2)
    gsems = (gs0, gs1, gs2)
    ssems = (ss0, ss1, ss2)

    def zrow(r2, _):
      for k in range(d // 16):
        fbuf0[r2, pl.ds(k * 16, 16)] = jnp.zeros((16,), jnp.float32)
      return 0

    lax.fori_loop(0, _B, zrow, 0)
    base = pl.multiple_of(sid * rpt, 8)
    nfull, rem = divmod(rpt, _B)
    for zi in range(nfull):
      pltpu.sync_copy(fbuf0, acc.at[pl.ds(base + zi * _B, _B)])
    if rem:
      pltpu.sync_copy(fbuf0.at[pl.ds(0, rem)],
                      acc.at[pl.ds(base + nfull * _B, rem)])
    plsc.subcore_barrier()

    def scale(fbuf, j):
      jv = jnp.full((16,), j, jnp.int32)

      def srow(r4, _):
        for u in range(4):
          r = r4 * 4 + u
          n16 = plsc.load_gather(
              norm_win, [jv, jnp.full((16,), r, jnp.int32)])
          for k in range(d // 16):
            sl = pl.ds(k * 16, 16)
            fbuf[r, sl] = fbuf[r, sl] * n16
        return 0

      lax.fori_loop(0, _B // 4, srow, 0)

    def win_loop(w, _):
      @pl.when(w > 0)
      def _():
        for p in range(3):
          pltpu.make_async_copy(
              fbufs[p], acc.at[idx_win.at[0, 1]], ssems[p]).wait()

      pltpu.sync_copy(idx_hbm.at[wid, pl.ds(w * _WB, _WB)], idx_win)
      pltpu.sync_copy(norm_hbm.at[wid, pl.ds(w * _WB, _WB)], norm_win)
      pltpu.async_copy(h_hbm.at[idx_win.at[0, 0]], fbuf0, gs0)

      def triple(j3, _):
        for b in range(3):
          j = j3 * 3 + b
          nj = j + 1
          q = (b + 1) % 3
          pltpu.make_async_copy(
              h_hbm.at[idx_win.at[j, 0]], fbufs[b], gsems[b]).wait()

          @pl.when(nj < _WB)
          def _():
            @pl.when(j >= 2)
            def _():
              pltpu.make_async_copy(
                  fbufs[q], acc.at[idx_win.at[0, 1]], ssems[q]).wait()

            pltpu.async_copy(h_hbm.at[idx_win.at[nj, 0]], fbufs[q], gsems[q])

          scale(fbufs[b], j)
          pltpu.async_copy(
              fbufs[b], acc.at[idx_win.at[j, 1]], ssems[b], add=True)
        return 0

      lax.fori_loop(0, _WB // 3, triple, 0)
      return 0

    lax.fori_loop(0, nwin, win_loop, 0)
    for p in range(3):
      pltpu.make_async_copy(
          fbufs[p], acc.at[idx_win.at[0, 1]], ssems[p]).wait()
    plsc.subcore_barrier()
    pltpu.sync_copy(acc.at[pl.ds(base, rpt)],
                    out_hbm.at[cid, pl.ds(base, rpt)])

  return hop_k


def _combine_tc(parts):
  _, n_pad, d = parts.shape
  blk = 1024

  def body(p_ref, o_ref):
    o_ref[...] = p_ref[0] + p_ref[1]

  return pl.pallas_call(
      body,
      grid=(n_pad // blk,),
      in_specs=[pl.BlockSpec((2, blk, d), lambda i: (0, i, 0))],
      out_specs=pl.BlockSpec((blk, d), lambda i: (i, 0)),
      out_shape=jax.ShapeDtypeStruct((n_pad, d), jnp.float32),
  )(parts)


def _final_tc(parts, w, b2d):
  _, n_pad, d = parts.shape
  c = w.shape[1]
  blk = 1024

  def body(p_ref, w_ref, b_ref, o_ref):
    h = p_ref[0] + p_ref[1]
    y = jnp.dot(h, w_ref[...], preferred_element_type=jnp.float32)
    y = y + b_ref[...]
    m = jnp.max(y, axis=1, keepdims=True)
    lse = jnp.log(jnp.sum(jnp.exp(y - m), axis=1, keepdims=True)) + m
    o_ref[...] = y - lse

  return pl.pallas_call(
      body,
      grid=(n_pad // blk,),
      in_specs=[
          pl.BlockSpec((2, blk, d), lambda i: (0, i, 0)),
          pl.BlockSpec((d, c), lambda i: (0, 0)),
          pl.BlockSpec((1, c), lambda i: (0, 0)),
      ],
      out_specs=pl.BlockSpec((blk, c), lambda i: (i, 0)),
      out_shape=jax.ShapeDtypeStruct((n_pad, c), jnp.float32),
  )(parts, w, b2d)


def kernel(x, edge_index, edge_attr, W, b):
  n, d = x.shape
  e = edge_index.shape[1]

  n_pad = -(-n // 2048) * 2048
  e_tot = e + n
  eb = _NW * _B
  nb = -(-e_tot // eb)
  nb = -(-nb // _WB) * _WB
  e_pad = nb * eb

  loop = jnp.arange(n, dtype=jnp.int32)
  pad = e_pad - e_tot
  pad_idx = jnp.arange(pad, dtype=jnp.int32) % n_pad
  row_p = jnp.concatenate([edge_index[0], loop, pad_idx]).reshape(_NW, nb, _B)
  col_p = jnp.concatenate([edge_index[1], loop, pad_idx]).reshape(_NW, nb, _B)
  w_p = jnp.concatenate([
      edge_attr.astype(jnp.float32),
      jnp.ones((n,), jnp.float32),
      jnp.zeros((pad,), jnp.float32),
  ]).reshape(_NW, nb, _B)

  x_pad = jnp.zeros((n_pad, d), jnp.float32).at[:n].set(x.astype(jnp.float32))

  deg_parts = _make_deg(n_pad, nb)(col_p, w_p)
  dis = _dis_tc(deg_parts.reshape(_NC, n_pad // 128, 128)).reshape(n_pad)
  norm_p = _make_norm(n_pad, nb)(row_p, col_p, w_p, dis)

  idx_p = jnp.stack([row_p, col_p], axis=2)

  hop = _make_hop(n_pad, d, nb)
  parts = hop(x_pad, idx_p, norm_p)
  h1 = _combine_tc(parts)
  parts2 = hop(h1, idx_p, norm_p)

  y = _final_tc(parts2, W.astype(jnp.float32), b.reshape(1, -1))
  return y[:n]


# speedup vs baseline: 1.1310x; 1.1310x over previous
import functools

import jax
import jax.numpy as jnp
import numpy as np
from jax import lax
from jax.experimental import pallas as pl
from jax.experimental.pallas import tpu as pltpu
from jax.experimental.pallas import tpu_sc as plsc

_NC = 2
_NS = 16
_NW = _NC * _NS
_B = 72
_DB = 128
_WB = 24


def _mesh():
  return plsc.VectorSubcoreMesh(core_axis_name="c", subcore_axis_name="s")


def _make_deg(n_pad, nb):
  rpt = n_pad // _NS

  @functools.partial(
      pl.kernel,
      out_type=jax.ShapeDtypeStruct((_NC, n_pad), jnp.float32),
      mesh=_mesh(),
      scratch_types=[
          pltpu.VMEM((nb, _DB), jnp.int32),
          pltpu.VMEM((nb, _DB), jnp.float32),
          pltpu.VMEM((_DB,), jnp.float32),
          pltpu.VMEM_SHARED((n_pad,), jnp.float32),
      ],
  )
  def deg_k(col_hbm, w_hbm, out_hbm, col_v, w_v, zbuf, acc):
    cid = lax.axis_index("c")
    sid = lax.axis_index("s")
    wid = cid * _NS + sid

    def zlane(i, _):
      zbuf[pl.ds(i * 16, 16)] = jnp.zeros((16,), jnp.float32)
      return 0

    lax.fori_loop(0, _DB // 16, zlane, 0)

    base = pl.multiple_of(sid * rpt, 8)
    nfull, rem = divmod(rpt, _DB)
    for zi in range(nfull):
      pltpu.sync_copy(zbuf, acc.at[pl.ds(base + zi * _DB, _DB)])
    if rem:
      pltpu.sync_copy(zbuf.at[pl.ds(0, rem)],
                      acc.at[pl.ds(base + nfull * _DB, rem)])
    pltpu.sync_copy(col_hbm.at[wid], col_v)
    pltpu.sync_copy(w_hbm.at[wid], w_v)
    plsc.subcore_barrier()

    def body(j, _):
      pltpu.sync_copy(w_v.at[j], acc.at[col_v.at[j]], add=True)
      return 0

    lax.fori_loop(0, nb, body, 0)
    plsc.subcore_barrier()
    pltpu.sync_copy(acc.at[pl.ds(base, rpt)], out_hbm.at[cid, pl.ds(base, rpt)])

  return deg_k


def _dis_tc(deg_parts2d):
  _, r, c = deg_parts2d.shape

  def body(p_ref, o_ref):
    deg = p_ref[0] + p_ref[1]
    o_ref[...] = jnp.where(
        deg > 0.0, lax.rsqrt(jnp.maximum(deg, 1e-12)), 0.0)

  return pl.pallas_call(
      body,
      out_shape=jax.ShapeDtypeStruct((r, c), jnp.float32),
  )(deg_parts2d)


def _make_norm(n_pad, ept):
  @functools.partial(
      pl.kernel,
      out_type=jax.ShapeDtypeStruct((_NW, ept), jnp.float32),
      mesh=_mesh(),
      compiler_params=pltpu.CompilerParams(needs_layout_passes=False),
      scratch_types=[
          pltpu.VMEM((ept,), jnp.int32),
          pltpu.VMEM((ept,), jnp.int32),
          pltpu.VMEM((ept,), jnp.float32),
          pltpu.VMEM((ept,), jnp.float32),
          pltpu.VMEM((n_pad,), jnp.float32),
      ],
  )
  def norm_k(row_hbm, col_hbm, w_hbm, dis_hbm, out_hbm,
             row_v, col_v, w_v, norm_v, dis_v):
    cid = lax.axis_index("c")
    sid = lax.axis_index("s")
    wid = cid * _NS + sid
    pltpu.sync_copy(row_hbm.at[wid], row_v)
    pltpu.sync_copy(col_hbm.at[wid], col_v)
    pltpu.sync_copy(w_hbm.at[wid], w_v)
    pltpu.sync_copy(dis_hbm, dis_v)

    def body(i, _):
      sl = pl.ds(i * 16, 16)
      a = plsc.load_gather(dis_v, [row_v[sl]])
      bb = plsc.load_gather(dis_v, [col_v[sl]])
      norm_v[sl] = a * w_v[sl] * bb
      return 0

    lax.fori_loop(0, ept // 16, body, 0)
    pltpu.sync_copy(norm_v, out_hbm.at[wid])

  return norm_k


def _make_hop(n_pad, d, nb):
  rpt = n_pad // _NS
  nwin = nb // _WB

  @functools.partial(
      pl.kernel,
      out_type=jax.ShapeDtypeStruct((_NC, n_pad, d), jnp.float32),
      mesh=_mesh(),
      compiler_params=pltpu.CompilerParams(needs_layout_passes=False),
      scratch_types=[
          pltpu.VMEM((_WB, 2, _B), jnp.int32),
          pltpu.VMEM((_WB, 2, _B), jnp.int32),
          pltpu.VMEM((_WB, _B), jnp.float32),
          pltpu.VMEM((_WB, _B), jnp.float32),
          pltpu.VMEM((_B, d), jnp.float32),
          pltpu.VMEM((_B, d), jnp.float32),
          pltpu.VMEM((_B, d), jnp.float32),
          pltpu.VMEM_SHARED((n_pad, d), jnp.float32),
          pltpu.SemaphoreType.DMA,
          pltpu.SemaphoreType.DMA,
          pltpu.SemaphoreType.DMA,
          pltpu.SemaphoreType.DMA,
          pltpu.SemaphoreType.DMA,
          pltpu.SemaphoreType.DMA,
          pltpu.SemaphoreType.DMA,
          pltpu.SemaphoreType.DMA,
          pltpu.SemaphoreType.DMA,
          pltpu.SemaphoreType.DMA,
      ],
  )
  def hop_k(h_hbm, idx_hbm, norm_hbm, out_hbm,
            idx_win0, idx_win1, norm_win0, norm_win1, fbuf0, fbuf1, fbuf2,
            acc, gs0, gs1, gs2, ss0, ss1, ss2, is0, is1, ns0, ns1):
    cid = lax.axis_index("c")
    sid = lax.axis_index("s")
    wid = cid * _NS + sid
    iwins = (idx_win0, idx_win1)
    nwins = (norm_win0, norm_win1)
    isems = (is0, is1)
    nsems = (ns0, ns1)
    fbufs = (fbuf0, fbuf1, fbuf2)
    gsems = (gs0, gs1, gs2)
    ssems = (ss0, ss1, ss2)

    def zrow(r2, _):
      for k in range(d // 16):
        fbuf0[r2, pl.ds(k * 16, 16)] = jnp.zeros((16,), jnp.float32)
      return 0

    lax.fori_loop(0, _B, zrow, 0)
    base = pl.multiple_of(sid * rpt, 8)
    nfull, rem = divmod(rpt, _B)
    for zi in range(nfull):
      pltpu.sync_copy(fbuf0, acc.at[pl.ds(base + zi * _B, _B)])
    if rem:
      pltpu.sync_copy(fbuf0.at[pl.ds(0, rem)],
                      acc.at[pl.ds(base + nfull * _B, rem)])
    plsc.subcore_barrier()

    def scale(fbuf, norm_win, j):
      jv = jnp.full((16,), j, jnp.int32)

      def srow(r4, _):
        for u in range(4):
          r = r4 * 4 + u
          n16 = plsc.load_gather(
              norm_win, [jv, jnp.full((16,), r, jnp.int32)])
          for k in range(d // 16):
            sl = pl.ds(k * 16, 16)
            fbuf[r, sl] = fbuf[r, sl] * n16
        return 0

      lax.fori_loop(0, _B // 4, srow, 0)

    pltpu.sync_copy(idx_hbm.at[wid, pl.ds(0, _WB)], idx_win0)
    pltpu.sync_copy(norm_hbm.at[wid, pl.ds(0, _WB)], norm_win0)
    pltpu.async_copy(h_hbm.at[idx_win0.at[0, 0]], fbuf0, gs0)

    def win_pair(w2, _):
      for p01 in range(2):
        w = w2 * 2 + p01
        iw, nw_ = iwins[p01], nwins[p01]
        ow, onw = iwins[1 - p01], nwins[1 - p01]

        def triple(j3, _):
          for b in range(3):
            j = j3 * 3 + b
            g = w * _WB + j
            nj = j + 1
            q = (b + 1) % 3
            pltpu.make_async_copy(
                h_hbm.at[iw.at[j, 0]], fbufs[b], gsems[b]).wait()

            @pl.when(g >= 2)
            def _():
              pltpu.make_async_copy(
                  fbufs[q], acc.at[iw.at[0, 1]], ssems[q]).wait()

            if b < 2:
              pltpu.async_copy(h_hbm.at[iw.at[nj, 0]], fbufs[q], gsems[q])
            else:
              @pl.when(nj < _WB)
              def _():
                pltpu.async_copy(h_hbm.at[iw.at[nj, 0]], fbufs[q], gsems[q])

              @pl.when(jnp.logical_and(nj == _WB, w + 1 < nwin))
              def _():
                pltpu.make_async_copy(
                    idx_hbm.at[wid, pl.ds((w + 1) * _WB, _WB)], ow,
                    isems[1 - p01]).wait()
                pltpu.make_async_copy(
                    norm_hbm.at[wid, pl.ds((w + 1) * _WB, _WB)], onw,
                    nsems[1 - p01]).wait()
                pltpu.async_copy(h_hbm.at[ow.at[0, 0]], fbufs[q], gsems[q])

            scale(fbufs[b], nw_, j)
            pltpu.async_copy(
                fbufs[b], acc.at[iw.at[j, 1]], ssems[b], add=True)

          @pl.when(jnp.logical_and(j3 == 0, w + 1 < nwin))
          def _():
            pltpu.async_copy(
                idx_hbm.at[wid, pl.ds((w + 1) * _WB, _WB)], ow,
                isems[1 - p01])
            pltpu.async_copy(
                norm_hbm.at[wid, pl.ds((w + 1) * _WB, _WB)], onw,
                nsems[1 - p01])
          return 0

        lax.fori_loop(0, _WB // 3, triple, 0)
      return 0

    lax.fori_loop(0, nwin // 2, win_pair, 0)
    for p in ((nb - 2) % 3, (nb - 1) % 3):
      pltpu.make_async_copy(
          fbufs[p], acc.at[idx_win0.at[0, 1]], ssems[p]).wait()
    plsc.subcore_barrier()
    pltpu.sync_copy(acc.at[pl.ds(base, rpt)],
                    out_hbm.at[cid, pl.ds(base, rpt)])

  return hop_k


def _combine_tc(parts):
  _, n_pad, d = parts.shape
  blk = 1024

  def body(p_ref, o_ref):
    o_ref[...] = p_ref[0] + p_ref[1]

  return pl.pallas_call(
      body,
      grid=(n_pad // blk,),
      in_specs=[pl.BlockSpec((2, blk, d), lambda i: (0, i, 0))],
      out_specs=pl.BlockSpec((blk, d), lambda i: (i, 0)),
      out_shape=jax.ShapeDtypeStruct((n_pad, d), jnp.float32),
  )(parts)


def _final_tc(parts, w, b2d):
  _, n_pad, d = parts.shape
  c = w.shape[1]
  blk = 1024

  def body(p_ref, w_ref, b_ref, o_ref):
    h = p_ref[0] + p_ref[1]
    y = jnp.dot(h, w_ref[...], preferred_element_type=jnp.float32)
    y = y + b_ref[...]
    m = jnp.max(y, axis=1, keepdims=True)
    lse = jnp.log(jnp.sum(jnp.exp(y - m), axis=1, keepdims=True)) + m
    o_ref[...] = y - lse

  return pl.pallas_call(
      body,
      grid=(n_pad // blk,),
      in_specs=[
          pl.BlockSpec((2, blk, d), lambda i: (0, i, 0)),
          pl.BlockSpec((d, c), lambda i: (0, 0)),
          pl.BlockSpec((1, c), lambda i: (0, 0)),
      ],
      out_specs=pl.BlockSpec((blk, c), lambda i: (i, 0)),
      out_shape=jax.ShapeDtypeStruct((n_pad, c), jnp.float32),
  )(parts, w, b2d)


def kernel(x, edge_index, edge_attr, W, b):
  n, d = x.shape
  e = edge_index.shape[1]

  n_pad = -(-n // 2048) * 2048
  e_tot = e + n
  eb = _NW * _B
  nb = -(-e_tot // eb)
  nb = -(-nb // (2 * _WB)) * (2 * _WB)
  e_pad = nb * eb
  ept = nb * _B
  nbd = ept // _DB

  loop = jnp.arange(n, dtype=jnp.int32)
  pad = e_pad - e_tot
  pad_idx = jnp.arange(pad, dtype=jnp.int32) % n_pad
  row_f = jnp.concatenate([edge_index[0], loop, pad_idx])
  col_f = jnp.concatenate([edge_index[1], loop, pad_idx])
  w_f = jnp.concatenate([
      edge_attr.astype(jnp.float32),
      jnp.ones((n,), jnp.float32),
      jnp.zeros((pad,), jnp.float32),
  ])

  x_pad = jnp.zeros((n_pad, d), jnp.float32).at[:n].set(x.astype(jnp.float32))

  deg_parts = _make_deg(n_pad, nbd)(
      col_f.reshape(_NW, nbd, _DB), w_f.reshape(_NW, nbd, _DB))
  dis = _dis_tc(deg_parts.reshape(_NC, n_pad // 128, 128)).reshape(n_pad)
  norm_flat = _make_norm(n_pad, ept)(
      row_f.reshape(_NW, ept), col_f.reshape(_NW, ept),
      w_f.reshape(_NW, ept), dis)

  idx_p = jnp.stack(
      [row_f.reshape(_NW, nb, _B), col_f.reshape(_NW, nb, _B)], axis=2)
  norm_p = norm_flat.reshape(_NW, nb, _B)

  hop = _make_hop(n_pad, d, nb)
  parts = hop(x_pad, idx_p, norm_p)
  h1 = _combine_tc(parts)
  parts2 = hop(h1, idx_p, norm_p)

  y = _final_tc(parts2, W.astype(jnp.float32), b.reshape(1, -1))
  return y[:n]


# speedup vs baseline: 1.1314x; 1.0003x over previous
import functools

import jax
import jax.numpy as jnp
from jax import lax
from jax.experimental import pallas as pl
from jax.experimental.pallas import tpu as pltpu
from jax.experimental.pallas import tpu_sc as plsc

_NC = 2
_NS = 16
_NW = _NC * _NS
_B = 72
_DB = 128
_WB = 24


def _mesh():
  return plsc.VectorSubcoreMesh(core_axis_name="c", subcore_axis_name="s")


def _make_deg(n_pad, nb):
  rpt = n_pad // _NS

  @functools.partial(
      pl.kernel,
      out_type=jax.ShapeDtypeStruct((_NC, n_pad), jnp.float32),
      mesh=_mesh(),
      scratch_types=[
          pltpu.VMEM((nb, _DB), jnp.int32),
          pltpu.VMEM((nb, _DB), jnp.float32),
          pltpu.VMEM((_DB,), jnp.float32),
          pltpu.VMEM_SHARED((n_pad,), jnp.float32),
      ],
  )
  def deg_k(col_hbm, w_hbm, out_hbm, col_v, w_v, zbuf, acc):
    cid = lax.axis_index("c")
    sid = lax.axis_index("s")
    wid = cid * _NS + sid

    def zlane(i, _):
      zbuf[pl.ds(i * 16, 16)] = jnp.zeros((16,), jnp.float32)
      return 0

    lax.fori_loop(0, _DB // 16, zlane, 0)

    base = pl.multiple_of(sid * rpt, 8)
    nfull, rem = divmod(rpt, _DB)
    for zi in range(nfull):
      pltpu.sync_copy(zbuf, acc.at[pl.ds(base + zi * _DB, _DB)])
    if rem:
      pltpu.sync_copy(zbuf.at[pl.ds(0, rem)],
                      acc.at[pl.ds(base + nfull * _DB, rem)])
    pltpu.sync_copy(col_hbm.at[wid], col_v)
    pltpu.sync_copy(w_hbm.at[wid], w_v)
    plsc.subcore_barrier()

    def body(j, _):
      pltpu.sync_copy(w_v.at[j], acc.at[col_v.at[j]], add=True)
      return 0

    lax.fori_loop(0, nb, body, 0)
    plsc.subcore_barrier()
    pltpu.sync_copy(acc.at[pl.ds(base, rpt)], out_hbm.at[cid, pl.ds(base, rpt)])

  return deg_k


def _dis_tc(deg_parts2d):
  _, r, c = deg_parts2d.shape

  def body(p_ref, o_ref):
    deg = p_ref[0] + p_ref[1]
    o_ref[...] = jnp.where(
        deg > 0.0, lax.rsqrt(jnp.maximum(deg, 1e-12)), 0.0)

  return pl.pallas_call(
      body,
      out_shape=jax.ShapeDtypeStruct((r, c), jnp.float32),
  )(deg_parts2d)


def _make_norm(n_pad, ept):
  @functools.partial(
      pl.kernel,
      out_type=jax.ShapeDtypeStruct((_NW, ept), jnp.float32),
      mesh=_mesh(),
      compiler_params=pltpu.CompilerParams(needs_layout_passes=False),
      scratch_types=[
          pltpu.VMEM((ept,), jnp.int32),
          pltpu.VMEM((ept,), jnp.int32),
          pltpu.VMEM((ept,), jnp.float32),
          pltpu.VMEM((ept,), jnp.float32),
          pltpu.VMEM((n_pad,), jnp.float32),
      ],
  )
  def norm_k(row_hbm, col_hbm, w_hbm, dis_hbm, out_hbm,
             row_v, col_v, w_v, norm_v, dis_v):
    cid = lax.axis_index("c")
    sid = lax.axis_index("s")
    wid = cid * _NS + sid
    pltpu.sync_copy(row_hbm.at[wid], row_v)
    pltpu.sync_copy(col_hbm.at[wid], col_v)
    pltpu.sync_copy(w_hbm.at[wid], w_v)
    pltpu.sync_copy(dis_hbm, dis_v)

    def body(i, _):
      sl = pl.ds(i * 16, 16)
      a = plsc.load_gather(dis_v, [row_v[sl]])
      bb = plsc.load_gather(dis_v, [col_v[sl]])
      norm_v[sl] = a * w_v[sl] * bb
      return 0

    lax.fori_loop(0, ept // 16, body, 0)
    pltpu.sync_copy(norm_v, out_hbm.at[wid])

  return norm_k


def _make_hop(n_pad, d, nb):
  rpt = n_pad // _NS
  nwin = nb // _WB

  @functools.partial(
      pl.kernel,
      out_type=jax.ShapeDtypeStruct((_NC, n_pad, d), jnp.float32),
      mesh=_mesh(),
      compiler_params=pltpu.CompilerParams(needs_layout_passes=False),
      scratch_types=[
          pltpu.VMEM((_WB, 2, _B), jnp.int32),
          pltpu.VMEM((_WB, 2, _B), jnp.int32),
          pltpu.VMEM((_WB, _B), jnp.float32),
          pltpu.VMEM((_WB, _B), jnp.float32),
          pltpu.VMEM((_B, d), jnp.float32),
          pltpu.VMEM((_B, d), jnp.float32),
          pltpu.VMEM((_B, d), jnp.float32),
          pltpu.VMEM_SHARED((n_pad, d), jnp.float32),
          pltpu.SemaphoreType.DMA,
          pltpu.SemaphoreType.DMA,
          pltpu.SemaphoreType.DMA,
          pltpu.SemaphoreType.DMA,
          pltpu.SemaphoreType.DMA,
          pltpu.SemaphoreType.DMA,
          pltpu.SemaphoreType.DMA,
          pltpu.SemaphoreType.DMA,
          pltpu.SemaphoreType.DMA,
          pltpu.SemaphoreType.DMA,
      ],
  )
  def hop_k(h_hbm, idx_hbm, norm_hbm, out_hbm,
            idx_win0, idx_win1, norm_win0, norm_win1, fbuf0, fbuf1, fbuf2,
            acc, gs0, gs1, gs2, ss0, ss1, ss2, is0, is1, ns0, ns1):
    cid = lax.axis_index("c")
    sid = lax.axis_index("s")
    wid = cid * _NS + sid
    iwins = (idx_win0, idx_win1)
    nwins = (norm_win0, norm_win1)
    isems = (is0, is1)
    nsems = (ns0, ns1)
    fbufs = (fbuf0, fbuf1, fbuf2)
    gsems = (gs0, gs1, gs2)
    ssems = (ss0, ss1, ss2)

    def zrow(r2, _):
      for k in range(d // 16):
        fbuf0[r2, pl.ds(k * 16, 16)] = jnp.zeros((16,), jnp.float32)
      return 0

    lax.fori_loop(0, _B, zrow, 0)
    base = pl.multiple_of(sid * rpt, 8)
    nfull, rem = divmod(rpt, _B)
    for zi in range(nfull):
      pltpu.sync_copy(fbuf0, acc.at[pl.ds(base + zi * _B, _B)])
    if rem:
      pltpu.sync_copy(fbuf0.at[pl.ds(0, rem)],
                      acc.at[pl.ds(base + nfull * _B, rem)])
    plsc.subcore_barrier()

    def scale(fbuf, norm_win, j):
      jv = jnp.full((16,), j, jnp.int32)

      def srow(r4, _):
        for u in range(4):
          r = r4 * 4 + u
          n16 = plsc.load_gather(
              norm_win, [jv, jnp.full((16,), r, jnp.int32)])
          for k in range(d // 16):
            sl = pl.ds(k * 16, 16)
            fbuf[r, sl] = fbuf[r, sl] * n16
        return 0

      lax.fori_loop(0, _B // 4, srow, 0)

    pltpu.sync_copy(idx_hbm.at[wid, pl.ds(0, _WB)], idx_win0)
    pltpu.sync_copy(norm_hbm.at[wid, pl.ds(0, _WB)], norm_win0)
    pltpu.async_copy(h_hbm.at[idx_win0.at[0, 0]], fbuf0, gs0)

    def win_pair(w2, _):
      for p01 in range(2):
        w = w2 * 2 + p01
        iw, nw_ = iwins[p01], nwins[p01]
        ow, onw = iwins[1 - p01], nwins[1 - p01]

        def triple(j3, _):
          for b in range(3):
            j = j3 * 3 + b
            g = w * _WB + j
            nj = j + 1
            q = (b + 1) % 3
            pltpu.make_async_copy(
                h_hbm.at[iw.at[j, 0]], fbufs[b], gsems[b]).wait()

            @pl.when(g >= 2)
            def _():
              pltpu.make_async_copy(
                  fbufs[q], acc.at[iw.at[0, 1]], ssems[q]).wait()

            if b < 2:
              pltpu.async_copy(h_hbm.at[iw.at[nj, 0]], fbufs[q], gsems[q])
            else:
              @pl.when(nj < _WB)
              def _():
                pltpu.async_copy(h_hbm.at[iw.at[nj, 0]], fbufs[q], gsems[q])

              @pl.when(jnp.logical_and(nj == _WB, w + 1 < nwin))
              def _():
                pltpu.make_async_copy(
                    idx_hbm.at[wid, pl.ds((w + 1) * _WB, _WB)], ow,
                    isems[1 - p01]).wait()
                pltpu.make_async_copy(
                    norm_hbm.at[wid, pl.ds((w + 1) * _WB, _WB)], onw,
                    nsems[1 - p01]).wait()
                pltpu.async_copy(h_hbm.at[ow.at[0, 0]], fbufs[q], gsems[q])

            scale(fbufs[b], nw_, j)
            pltpu.async_copy(
                fbufs[b], acc.at[iw.at[j, 1]], ssems[b], add=True)

          @pl.when(jnp.logical_and(j3 == 0, w + 1 < nwin))
          def _():
            pltpu.async_copy(
                idx_hbm.at[wid, pl.ds((w + 1) * _WB, _WB)], ow,
                isems[1 - p01])
            pltpu.async_copy(
                norm_hbm.at[wid, pl.ds((w + 1) * _WB, _WB)], onw,
                nsems[1 - p01])
          return 0

        lax.fori_loop(0, _WB // 3, triple, 0)
      return 0

    lax.fori_loop(0, nwin // 2, win_pair, 0)
    for p in ((nb - 2) % 3, (nb - 1) % 3):
      pltpu.make_async_copy(
          fbufs[p], acc.at[idx_win0.at[0, 1]], ssems[p]).wait()
    plsc.subcore_barrier()
    pltpu.sync_copy(acc.at[pl.ds(base, rpt)],
                    out_hbm.at[cid, pl.ds(base, rpt)])

  return hop_k


def _combine_tc(parts):
  _, n_pad, d = parts.shape
  blk = 1024

  def body(p_ref, o_ref):
    o_ref[...] = p_ref[0] + p_ref[1]

  return pl.pallas_call(
      body,
      grid=(n_pad // blk,),
      in_specs=[pl.BlockSpec((2, blk, d), lambda i: (0, i, 0))],
      out_specs=pl.BlockSpec((blk, d), lambda i: (i, 0)),
      out_shape=jax.ShapeDtypeStruct((n_pad, d), jnp.float32),
  )(parts)


def _final_tc(parts, w, b2d):
  _, n_pad, d = parts.shape
  c = w.shape[1]
  blk = 1024

  def body(p_ref, w_ref, b_ref, o_ref):
    h = p_ref[0] + p_ref[1]
    y = jnp.dot(h, w_ref[...], preferred_element_type=jnp.float32)
    y = y + b_ref[...]
    m = jnp.max(y, axis=1, keepdims=True)
    lse = jnp.log(jnp.sum(jnp.exp(y - m), axis=1, keepdims=True)) + m
    o_ref[...] = y - lse

  return pl.pallas_call(
      body,
      grid=(n_pad // blk,),
      in_specs=[
          pl.BlockSpec((2, blk, d), lambda i: (0, i, 0)),
          pl.BlockSpec((d, c), lambda i: (0, 0)),
          pl.BlockSpec((1, c), lambda i: (0, 0)),
      ],
      out_specs=pl.BlockSpec((blk, c), lambda i: (i, 0)),
      out_shape=jax.ShapeDtypeStruct((n_pad, c), jnp.float32),
  )(parts, w, b2d)


def kernel(x, edge_index, edge_attr, W, b):
  n, d = x.shape
  e = edge_index.shape[1]

  n_pad = -(-n // 2048) * 2048
  e_tot = e + n
  eb = _NW * _B
  nb = -(-e_tot // eb)
  nb = -(-nb // (2 * _WB)) * (2 * _WB)
  e_pad = nb * eb
  ept = nb * _B
  nbd = ept // _DB

  loop = jnp.arange(n, dtype=jnp.int32)
  pad = e_pad - e_tot
  pad_idx = jnp.arange(pad, dtype=jnp.int32) % n_pad
  row_f = jnp.concatenate([edge_index[0], loop, pad_idx])
  col_f = jnp.concatenate([edge_index[1], loop, pad_idx])
  w_f = jnp.concatenate([
      edge_attr.astype(jnp.float32),
      jnp.ones((n,), jnp.float32),
      jnp.zeros((pad,), jnp.float32),
  ])

  x_pad = jnp.zeros((n_pad, d), jnp.float32).at[:n].set(x.astype(jnp.float32))

  deg_parts = _make_deg(n_pad, nbd)(
      col_f.reshape(_NW, nbd, _DB), w_f.reshape(_NW, nbd, _DB))
  dis = _dis_tc(deg_parts.reshape(_NC, n_pad // 128, 128)).reshape(n_pad)
  norm_flat = _make_norm(n_pad, ept)(
      row_f.reshape(_NW, ept), col_f.reshape(_NW, ept),
      w_f.reshape(_NW, ept), dis)

  idx_p = jnp.stack(
      [row_f.reshape(_NW, nb, _B), col_f.reshape(_NW, nb, _B)], axis=2)
  norm_p = norm_flat.reshape(_NW, nb, _B)

  hop = _make_hop(n_pad, d, nb)
  parts = hop(x_pad, idx_p, norm_p)
  h1 = _combine_tc(parts)
  parts2 = hop(h1, idx_p, norm_p)

  y = _final_tc(parts2, W.astype(jnp.float32), b.reshape(1, -1))
  return y[:n]
